# baseline (speedup 1.0000x reference)
import jax
import jax.numpy as jnp
import numpy as np
from jax import lax
from jax.experimental import pallas as pl
from jax.experimental.pallas import tpu as pltpu

_V = 50257
_B = 64
_D = 2048
_NS = 100
_VT = 512
_NT = (_V + _VT - 1) // _VT
_VPAD = _NT * _VT

_R0 = (13, 15, 26, 6)
_R1 = (17, 29, 16, 24)

_LO = np.float32(np.nextafter(np.float32(-1.0), np.float32(0.0)))
_TINY = np.float32(np.finfo(np.float32).tiny)
_SQRT2 = np.float32(np.sqrt(2.0))
_NEG = np.float32(-1e30)


def _rotl(x, d):
    return lax.bitwise_or(lax.shift_left(x, np.int32(d)),
                          lax.shift_right_logical(x, np.int32(32 - d)))


def _rounds(x0, x1, rots):
    for r in rots:
        x0 = x0 + x1
        x1 = _rotl(x1, r)
        x1 = lax.bitwise_xor(x0, x1)
    return x0, x1


def _threefry_xor(k0, k1, c1):
    ks2 = lax.bitwise_xor(lax.bitwise_xor(k0, k1), np.int32(0x1BD11BDA))
    x0 = jnp.zeros_like(c1) + k0
    x1 = c1 + k1
    x0, x1 = _rounds(x0, x1, _R0); x0 = x0 + k1; x1 = x1 + ks2 + np.int32(1)
    x0, x1 = _rounds(x0, x1, _R1); x0 = x0 + ks2; x1 = x1 + k0 + np.int32(2)
    x0, x1 = _rounds(x0, x1, _R0); x0 = x0 + k0; x1 = x1 + k1 + np.int32(3)
    x0, x1 = _rounds(x0, x1, _R1); x0 = x0 + k1; x1 = x1 + ks2 + np.int32(4)
    x0, x1 = _rounds(x0, x1, _R0); x0 = x0 + ks2; x1 = x1 + k0 + np.int32(5)
    return lax.bitwise_xor(x0, x1)


def _u01(bits):
    fb = lax.bitwise_or(lax.shift_right_logical(bits, np.int32(9)),
                        np.int32(0x3F800000))
    return lax.bitcast_convert_type(fb, jnp.float32) - np.float32(1.0)


def _erfinv(x):
    w = -jnp.log1p(-x * x)
    small = w < np.float32(5.0)
    ws = w - np.float32(2.5)
    wb = jnp.sqrt(w) - np.float32(3.0)
    p1 = np.float32(2.81022636e-08)
    for c in (3.43273939e-07, -3.5233877e-06, -4.39150654e-06, 0.00021858087,
              -0.00125372503, -0.00417768164, 0.246640727, 1.50140941):
        p1 = p1 * ws + np.float32(c)
    p2 = np.float32(-0.000200214257)
    for c in (0.000100950558, 0.00134934322, -0.00367342844, 0.00573950773,
              -0.0076224613, 0.00943887047, 1.00167406, 2.83297682):
        p2 = p2 * wb + np.float32(c)
    return jnp.where(small, p1, p2) * x


def _normal(bits):
    f = _u01(bits)
    u = f * (np.float32(1.0) - _LO) + _LO
    u = jnp.maximum(_LO, u)
    return _SQRT2 * _erfinv(u)


def _gumbel(bits):
    f = _u01(bits)
    u = f * (np.float32(1.0) - _TINY) + _TINY
    u = jnp.maximum(_TINY, u)
    return -jnp.log(-jnp.log(u))


def _sample_kernel(h_ref, w_ref, b_ref, kn_ref, kg_ref, t_ref,
                   val_ref, idx_ref, logits_ref):
    i = pl.program_id(0)
    j = pl.program_id(1)

    @pl.when(j == 0)
    def _matmul():
        acc = lax.dot_general(h_ref[...], w_ref[...],
                              (((1,), (1,)), ((), ())),
                              preferred_element_type=jnp.float32)
        logits_ref[...] = acc + b_ref[0, :][None, :]

    col = lax.broadcasted_iota(jnp.int32, (_B, _VT), 1) + i * np.int32(_VT)
    row = lax.broadcasted_iota(jnp.int32, (_B, _VT), 0)
    cnt = row * np.int32(_V) + col

    n = _normal(_threefry_xor(kn_ref[j, 0], kn_ref[j, 1], cnt))
    g = _gumbel(_threefry_xor(kg_ref[j, 0], kg_ref[j, 1], cnt))

    t = t_ref[j, 0]
    score = (logits_ref[...] + n * t) + g

    m = jnp.max(score, axis=1, keepdims=True)
    cand = jnp.where(score == m, col, np.int32(0x7FFFFFFF))
    ix = jnp.min(cand, axis=1, keepdims=True)

    val_ref[...] = m.reshape(1, 1, 1, _B)
    idx_ref[...] = ix.reshape(1, 1, 1, _B)


def _merge_mode_kernel(val_ref, idx_ref, out_ref):
    bv = val_ref[0, :, 0, :]
    bi = idx_ref[0, :, 0, :]
    for k in range(1, _NT):
        v = val_ref[k, :, 0, :]
        ixk = idx_ref[k, :, 0, :]
        better = v > bv
        bv = jnp.where(better, v, bv)
        bi = jnp.where(better, ixk, bi)
    counts = jnp.zeros((_NS, _B), dtype=jnp.int32)
    for s in range(_NS):
        counts = counts + (bi == bi[s:s + 1, :]).astype(jnp.int32)
    maxc = jnp.max(counts, axis=0, keepdims=True)
    mode = jnp.min(jnp.where(counts == maxc, bi, np.int32(_V)),
                   axis=0, keepdims=True)
    out_ref[...] = mode


def kernel(h, W, b):
    Wp = jnp.pad(W, ((0, _VPAD - _V), (0, 0)))
    bp = jnp.pad(b, (0, _VPAD - _V), constant_values=_NEG).reshape(1, _VPAD)

    root = jax.random.key(1)
    step_keys = jax.random.split(root, _NS)
    sub = jax.vmap(jax.random.split)(step_keys)
    kn = lax.bitcast_convert_type(jax.random.key_data(sub[:, 0]), jnp.int32)
    kg = lax.bitcast_convert_type(jax.random.key_data(sub[:, 1]), jnp.int32)
    temps = jnp.linspace(1.0, 0.01, _NS).astype(jnp.float32).reshape(_NS, 1)

    val, idx = pl.pallas_call(
        _sample_kernel,
        grid=(_NT, _NS),
        in_specs=[
            pl.BlockSpec((_B, _D), lambda i, j: (0, 0)),
            pl.BlockSpec((_VT, _D), lambda i, j: (i, 0)),
            pl.BlockSpec((1, _VT), lambda i, j: (0, i)),
            pl.BlockSpec((_NS, 2), lambda i, j: (0, 0), memory_space=pltpu.SMEM),
            pl.BlockSpec((_NS, 2), lambda i, j: (0, 0), memory_space=pltpu.SMEM),
            pl.BlockSpec((_NS, 1), lambda i, j: (0, 0), memory_space=pltpu.SMEM),
        ],
        out_specs=[
            pl.BlockSpec((1, 1, 1, _B), lambda i, j: (i, j, 0, 0)),
            pl.BlockSpec((1, 1, 1, _B), lambda i, j: (i, j, 0, 0)),
        ],
        out_shape=[
            jax.ShapeDtypeStruct((_NT, _NS, 1, _B), jnp.float32),
            jax.ShapeDtypeStruct((_NT, _NS, 1, _B), jnp.int32),
        ],
        scratch_shapes=[pltpu.VMEM((_B, _VT), jnp.float32)],
    )(h, Wp, bp, kn, kg, temps)

    out = pl.pallas_call(
        _merge_mode_kernel,
        out_shape=jax.ShapeDtypeStruct((1, _B), jnp.int32),
    )(val, idx)
    return out.reshape(_B)


# speedup vs baseline: 1.0860x; 1.0860x over previous
import jax
import jax.numpy as jnp
import numpy as np
from jax import lax
from jax.experimental import pallas as pl
from jax.experimental.pallas import tpu as pltpu

_V = 50257
_B = 64
_D = 2048
_NS = 100
_VT = 1024
_NT = (_V + _VT - 1) // _VT
_VPAD = _NT * _VT

_R0 = (13, 15, 26, 6)
_R1 = (17, 29, 16, 24)

_LO = np.float32(np.nextafter(np.float32(-1.0), np.float32(0.0)))
_TINY = np.float32(np.finfo(np.float32).tiny)
_SQRT2 = np.float32(np.sqrt(2.0))
_NEG = np.float32(-1e30)


def _rotl(x, d):
    return lax.bitwise_or(lax.shift_left(x, np.int32(d)),
                          lax.shift_right_logical(x, np.int32(32 - d)))


def _rounds(x0, x1, rots):
    for r in rots:
        x0 = x0 + x1
        x1 = _rotl(x1, r)
        x1 = lax.bitwise_xor(x0, x1)
    return x0, x1


def _threefry_xor(k0, k1, c1):
    ks2 = lax.bitwise_xor(lax.bitwise_xor(k0, k1), np.int32(0x1BD11BDA))
    x0 = jnp.zeros_like(c1) + k0
    x1 = c1 + k1
    x0, x1 = _rounds(x0, x1, _R0); x0 = x0 + k1; x1 = x1 + ks2 + np.int32(1)
    x0, x1 = _rounds(x0, x1, _R1); x0 = x0 + ks2; x1 = x1 + k0 + np.int32(2)
    x0, x1 = _rounds(x0, x1, _R0); x0 = x0 + k0; x1 = x1 + k1 + np.int32(3)
    x0, x1 = _rounds(x0, x1, _R1); x0 = x0 + k1; x1 = x1 + ks2 + np.int32(4)
    x0, x1 = _rounds(x0, x1, _R0); x0 = x0 + ks2; x1 = x1 + k0 + np.int32(5)
    return lax.bitwise_xor(x0, x1)


def _u01(bits):
    fb = lax.bitwise_or(lax.shift_right_logical(bits, np.int32(9)),
                        np.int32(0x3F800000))
    return lax.bitcast_convert_type(fb, jnp.float32) - np.float32(1.0)


def _erfinv(x):
    w = -jnp.log1p(-x * x)
    small = w < np.float32(5.0)
    ws = w - np.float32(2.5)
    wb = jnp.sqrt(w) - np.float32(3.0)
    p1 = np.float32(2.81022636e-08)
    for c in (3.43273939e-07, -3.5233877e-06, -4.39150654e-06, 0.00021858087,
              -0.00125372503, -0.00417768164, 0.246640727, 1.50140941):
        p1 = p1 * ws + np.float32(c)
    p2 = np.float32(-0.000200214257)
    for c in (0.000100950558, 0.00134934322, -0.00367342844, 0.00573950773,
              -0.0076224613, 0.00943887047, 1.00167406, 2.83297682):
        p2 = p2 * wb + np.float32(c)
    return jnp.where(small, p1, p2) * x


def _normal(bits):
    f = _u01(bits)
    u = f * (np.float32(1.0) - _LO) + _LO
    u = jnp.maximum(_LO, u)
    return _SQRT2 * _erfinv(u)


def _gumbel(bits):
    f = _u01(bits)
    u = f * (np.float32(1.0) - _TINY) + _TINY
    u = jnp.maximum(_TINY, u)
    return -jnp.log(-jnp.log(u))


def _sample_kernel(h_ref, w_ref, b_ref, kn_ref, kg_ref, t_ref,
                   val_ref, idx_ref, logits_ref):
    i = pl.program_id(0)
    j = pl.program_id(1)

    @pl.when(j == 0)
    def _matmul():
        acc = lax.dot_general(h_ref[...], w_ref[...],
                              (((1,), (1,)), ((), ())),
                              preferred_element_type=jnp.float32)
        logits_ref[...] = acc + b_ref[0, :][None, :]

    col = lax.broadcasted_iota(jnp.int32, (_B, _VT), 1) + i * np.int32(_VT)
    row = lax.broadcasted_iota(jnp.int32, (_B, _VT), 0)
    cnt = row * np.int32(_V) + col

    n = _normal(_threefry_xor(kn_ref[j, 0], kn_ref[j, 1], cnt))
    g = _gumbel(_threefry_xor(kg_ref[j, 0], kg_ref[j, 1], cnt))

    t = t_ref[j, 0]
    score = (logits_ref[...] + n * t) + g

    m = jnp.max(score, axis=1, keepdims=True)
    cand = jnp.where(score == m, col, np.int32(0x7FFFFFFF))
    ix = jnp.min(cand, axis=1, keepdims=True)

    val_ref[...] = m.reshape(1, 1, 1, _B)
    idx_ref[...] = ix.reshape(1, 1, 1, _B)


def _merge_mode_kernel(val_ref, idx_ref, out_ref):
    bv = val_ref[0, :, 0, :]
    bi = idx_ref[0, :, 0, :]
    for k in range(1, _NT):
        v = val_ref[k, :, 0, :]
        ixk = idx_ref[k, :, 0, :]
        better = v > bv
        bv = jnp.where(better, v, bv)
        bi = jnp.where(better, ixk, bi)
    counts = jnp.zeros((_NS, _B), dtype=jnp.int32)
    for s in range(_NS):
        counts = counts + (bi == bi[s:s + 1, :]).astype(jnp.int32)
    maxc = jnp.max(counts, axis=0, keepdims=True)
    mode = jnp.min(jnp.where(counts == maxc, bi, np.int32(_V)),
                   axis=0, keepdims=True)
    out_ref[...] = mode


def kernel(h, W, b):
    Wp = jnp.pad(W, ((0, _VPAD - _V), (0, 0)))
    bp = jnp.pad(b, (0, _VPAD - _V), constant_values=_NEG).reshape(1, _VPAD)

    root = jax.random.key(1)
    step_keys = jax.random.split(root, _NS)
    sub = jax.vmap(jax.random.split)(step_keys)
    kn = lax.bitcast_convert_type(jax.random.key_data(sub[:, 0]), jnp.int32)
    kg = lax.bitcast_convert_type(jax.random.key_data(sub[:, 1]), jnp.int32)
    temps = jnp.linspace(1.0, 0.01, _NS).astype(jnp.float32).reshape(_NS, 1)

    val, idx = pl.pallas_call(
        _sample_kernel,
        grid=(_NT, _NS),
        in_specs=[
            pl.BlockSpec((_B, _D), lambda i, j: (0, 0)),
            pl.BlockSpec((_VT, _D), lambda i, j: (i, 0)),
            pl.BlockSpec((1, _VT), lambda i, j: (0, i)),
            pl.BlockSpec((_NS, 2), lambda i, j: (0, 0), memory_space=pltpu.SMEM),
            pl.BlockSpec((_NS, 2), lambda i, j: (0, 0), memory_space=pltpu.SMEM),
            pl.BlockSpec((_NS, 1), lambda i, j: (0, 0), memory_space=pltpu.SMEM),
        ],
        out_specs=[
            pl.BlockSpec((1, 1, 1, _B), lambda i, j: (i, j, 0, 0)),
            pl.BlockSpec((1, 1, 1, _B), lambda i, j: (i, j, 0, 0)),
        ],
        out_shape=[
            jax.ShapeDtypeStruct((_NT, _NS, 1, _B), jnp.float32),
            jax.ShapeDtypeStruct((_NT, _NS, 1, _B), jnp.int32),
        ],
        scratch_shapes=[pltpu.VMEM((_B, _VT), jnp.float32)],
    )(h, Wp, bp, kn, kg, temps)

    out = pl.pallas_call(
        _merge_mode_kernel,
        out_shape=jax.ShapeDtypeStruct((1, _B), jnp.int32),
    )(val, idx)
    return out.reshape(_B)


# speedup vs baseline: 1.1029x; 1.0155x over previous
import jax
import jax.numpy as jnp
import numpy as np
from jax import lax
from jax.experimental import pallas as pl
from jax.experimental.pallas import tpu as pltpu

_V = 50257
_B = 64
_D = 2048
_NS = 100
_VT = 2048
_NT = (_V + _VT - 1) // _VT
_VPAD = _NT * _VT

_R0 = (13, 15, 26, 6)
_R1 = (17, 29, 16, 24)

_LO = np.float32(np.nextafter(np.float32(-1.0), np.float32(0.0)))
_TINY = np.float32(np.finfo(np.float32).tiny)
_SQRT2 = np.float32(np.sqrt(2.0))
_NEG = np.float32(-1e30)


def _rotl(x, d):
    return lax.bitwise_or(lax.shift_left(x, np.int32(d)),
                          lax.shift_right_logical(x, np.int32(32 - d)))


def _rounds(x0, x1, rots):
    for r in rots:
        x0 = x0 + x1
        x1 = _rotl(x1, r)
        x1 = lax.bitwise_xor(x0, x1)
    return x0, x1


def _threefry_xor(k0, k1, c1):
    ks2 = lax.bitwise_xor(lax.bitwise_xor(k0, k1), np.int32(0x1BD11BDA))
    x0 = jnp.zeros_like(c1) + k0
    x1 = c1 + k1
    x0, x1 = _rounds(x0, x1, _R0); x0 = x0 + k1; x1 = x1 + ks2 + np.int32(1)
    x0, x1 = _rounds(x0, x1, _R1); x0 = x0 + ks2; x1 = x1 + k0 + np.int32(2)
    x0, x1 = _rounds(x0, x1, _R0); x0 = x0 + k0; x1 = x1 + k1 + np.int32(3)
    x0, x1 = _rounds(x0, x1, _R1); x0 = x0 + k1; x1 = x1 + ks2 + np.int32(4)
    x0, x1 = _rounds(x0, x1, _R0); x0 = x0 + ks2; x1 = x1 + k0 + np.int32(5)
    return lax.bitwise_xor(x0, x1)


def _u01(bits):
    fb = lax.bitwise_or(lax.shift_right_logical(bits, np.int32(9)),
                        np.int32(0x3F800000))
    return lax.bitcast_convert_type(fb, jnp.float32) - np.float32(1.0)


def _erfinv(x):
    w = -jnp.log1p(-x * x)
    small = w < np.float32(5.0)
    ws = w - np.float32(2.5)
    wb = jnp.sqrt(w) - np.float32(3.0)
    p1 = np.float32(2.81022636e-08)
    for c in (3.43273939e-07, -3.5233877e-06, -4.39150654e-06, 0.00021858087,
              -0.00125372503, -0.00417768164, 0.246640727, 1.50140941):
        p1 = p1 * ws + np.float32(c)
    p2 = np.float32(-0.000200214257)
    for c in (0.000100950558, 0.00134934322, -0.00367342844, 0.00573950773,
              -0.0076224613, 0.00943887047, 1.00167406, 2.83297682):
        p2 = p2 * wb + np.float32(c)
    return jnp.where(small, p1, p2) * x


def _normal(bits):
    f = _u01(bits)
    u = f * (np.float32(1.0) - _LO) + _LO
    u = jnp.maximum(_LO, u)
    return _SQRT2 * _erfinv(u)


def _gumbel(bits):
    f = _u01(bits)
    u = f * (np.float32(1.0) - _TINY) + _TINY
    u = jnp.maximum(_TINY, u)
    return -jnp.log(-jnp.log(u))


def _sample_kernel(h_ref, w_ref, b_ref, kn_ref, kg_ref, t_ref,
                   val_ref, idx_ref, logits_ref):
    i = pl.program_id(0)
    j = pl.program_id(1)

    @pl.when(j == 0)
    def _matmul():
        acc = lax.dot_general(h_ref[...], w_ref[...],
                              (((1,), (1,)), ((), ())),
                              preferred_element_type=jnp.float32)
        logits_ref[...] = acc + b_ref[0, :][None, :]

    col = lax.broadcasted_iota(jnp.int32, (_B, _VT), 1) + i * np.int32(_VT)
    row = lax.broadcasted_iota(jnp.int32, (_B, _VT), 0)
    cnt = row * np.int32(_V) + col

    n = _normal(_threefry_xor(kn_ref[j, 0], kn_ref[j, 1], cnt))
    g = _gumbel(_threefry_xor(kg_ref[j, 0], kg_ref[j, 1], cnt))

    t = t_ref[j, 0]
    score = (logits_ref[...] + n * t) + g

    m = jnp.max(score, axis=1, keepdims=True)
    cand = jnp.where(score == m, col, np.int32(0x7FFFFFFF))
    ix = jnp.min(cand, axis=1, keepdims=True)

    val_ref[...] = m.reshape(1, 1, 1, _B)
    idx_ref[...] = ix.reshape(1, 1, 1, _B)


def _merge_mode_kernel(val_ref, idx_ref, out_ref):
    bv = val_ref[0, :, 0, :]
    bi = idx_ref[0, :, 0, :]
    for k in range(1, _NT):
        v = val_ref[k, :, 0, :]
        ixk = idx_ref[k, :, 0, :]
        better = v > bv
        bv = jnp.where(better, v, bv)
        bi = jnp.where(better, ixk, bi)
    counts = jnp.zeros((_NS, _B), dtype=jnp.int32)
    for s in range(_NS):
        counts = counts + (bi == bi[s:s + 1, :]).astype(jnp.int32)
    maxc = jnp.max(counts, axis=0, keepdims=True)
    mode = jnp.min(jnp.where(counts == maxc, bi, np.int32(_V)),
                   axis=0, keepdims=True)
    out_ref[...] = mode


def kernel(h, W, b):
    Wp = jnp.pad(W, ((0, _VPAD - _V), (0, 0)))
    bp = jnp.pad(b, (0, _VPAD - _V), constant_values=_NEG).reshape(1, _VPAD)

    root = jax.random.key(1)
    step_keys = jax.random.split(root, _NS)
    sub = jax.vmap(jax.random.split)(step_keys)
    kn = lax.bitcast_convert_type(jax.random.key_data(sub[:, 0]), jnp.int32)
    kg = lax.bitcast_convert_type(jax.random.key_data(sub[:, 1]), jnp.int32)
    temps = jnp.linspace(1.0, 0.01, _NS).astype(jnp.float32).reshape(_NS, 1)

    val, idx = pl.pallas_call(
        _sample_kernel,
        grid=(_NT, _NS),
        in_specs=[
            pl.BlockSpec((_B, _D), lambda i, j: (0, 0)),
            pl.BlockSpec((_VT, _D), lambda i, j: (i, 0)),
            pl.BlockSpec((1, _VT), lambda i, j: (0, i)),
            pl.BlockSpec((_NS, 2), lambda i, j: (0, 0), memory_space=pltpu.SMEM),
            pl.BlockSpec((_NS, 2), lambda i, j: (0, 0), memory_space=pltpu.SMEM),
            pl.BlockSpec((_NS, 1), lambda i, j: (0, 0), memory_space=pltpu.SMEM),
        ],
        out_specs=[
            pl.BlockSpec((1, 1, 1, _B), lambda i, j: (i, j, 0, 0)),
            pl.BlockSpec((1, 1, 1, _B), lambda i, j: (i, j, 0, 0)),
        ],
        out_shape=[
            jax.ShapeDtypeStruct((_NT, _NS, 1, _B), jnp.float32),
            jax.ShapeDtypeStruct((_NT, _NS, 1, _B), jnp.int32),
        ],
        scratch_shapes=[pltpu.VMEM((_B, _VT), jnp.float32)],
    )(h, Wp, bp, kn, kg, temps)

    out = pl.pallas_call(
        _merge_mode_kernel,
        out_shape=jax.ShapeDtypeStruct((1, _B), jnp.int32),
    )(val, idx)
    return out.reshape(_B)


# speedup vs baseline: 1.1081x; 1.0048x over previous
import jax
import jax.numpy as jnp
import numpy as np
from jax import lax
from jax.experimental import pallas as pl
from jax.experimental.pallas import tpu as pltpu

_V = 50257
_B = 64
_D = 2048
_NS = 100
_VT = 2048
_NT = (_V + _VT - 1) // _VT
_VPAD = _NT * _VT

_R0 = (13, 15, 26, 6)
_R1 = (17, 29, 16, 24)

_LO = np.float32(np.nextafter(np.float32(-1.0), np.float32(0.0)))
_TINY = np.float32(np.finfo(np.float32).tiny)
_SQRT2 = np.float32(np.sqrt(2.0))
_NEG = np.float32(-1e30)


def _rotl(x, d):
    return lax.bitwise_or(lax.shift_left(x, np.int32(d)),
                          lax.shift_right_logical(x, np.int32(32 - d)))


def _rounds(x0, x1, rots):
    for r in rots:
        x0 = x0 + x1
        x1 = _rotl(x1, r)
        x1 = lax.bitwise_xor(x0, x1)
    return x0, x1


def _threefry_xor(k0, k1, c1):
    ks2 = lax.bitwise_xor(lax.bitwise_xor(k0, k1), np.int32(0x1BD11BDA))
    x0 = jnp.zeros_like(c1) + k0
    x1 = c1 + k1
    x0, x1 = _rounds(x0, x1, _R0); x0 = x0 + k1; x1 = x1 + ks2 + np.int32(1)
    x0, x1 = _rounds(x0, x1, _R1); x0 = x0 + ks2; x1 = x1 + k0 + np.int32(2)
    x0, x1 = _rounds(x0, x1, _R0); x0 = x0 + k0; x1 = x1 + k1 + np.int32(3)
    x0, x1 = _rounds(x0, x1, _R1); x0 = x0 + k1; x1 = x1 + ks2 + np.int32(4)
    x0, x1 = _rounds(x0, x1, _R0); x0 = x0 + ks2; x1 = x1 + k0 + np.int32(5)
    return lax.bitwise_xor(x0, x1)


def _u01(bits):
    fb = lax.bitwise_or(lax.shift_right_logical(bits, np.int32(9)),
                        np.int32(0x3F800000))
    return lax.bitcast_convert_type(fb, jnp.float32) - np.float32(1.0)


def _erfinv(x):
    w = -jnp.log1p(-x * x)
    small = w < np.float32(5.0)
    ws = w - np.float32(2.5)
    wb = jnp.sqrt(w) - np.float32(3.0)
    p1 = np.float32(2.81022636e-08)
    for c in (3.43273939e-07, -3.5233877e-06, -4.39150654e-06, 0.00021858087,
              -0.00125372503, -0.00417768164, 0.246640727, 1.50140941):
        p1 = p1 * ws + np.float32(c)
    p2 = np.float32(-0.000200214257)
    for c in (0.000100950558, 0.00134934322, -0.00367342844, 0.00573950773,
              -0.0076224613, 0.00943887047, 1.00167406, 2.83297682):
        p2 = p2 * wb + np.float32(c)
    return jnp.where(small, p1, p2) * x


def _normal(bits):
    f = _u01(bits)
    u = f * (np.float32(1.0) - _LO) + _LO
    u = jnp.maximum(_LO, u)
    return _SQRT2 * _erfinv(u)


def _gumbel(bits):
    f = _u01(bits)
    u = f * (np.float32(1.0) - _TINY) + _TINY
    u = jnp.maximum(_TINY, u)
    return -jnp.log(-jnp.log(u))


def _sample_kernel(h_ref, w_ref, b_ref, kn_ref, kg_ref, t_ref,
                   val_ref, idx_ref, logits_ref, cnt_ref):
    i = pl.program_id(0)
    j = pl.program_id(1)

    @pl.when(j == 0)
    def _matmul():
        acc = lax.dot_general(h_ref[...], w_ref[...],
                              (((1,), (1,)), ((), ())),
                              preferred_element_type=jnp.float32)
        logits_ref[...] = acc + b_ref[0, :][None, :]
        col = lax.broadcasted_iota(jnp.int32, (_B, _VT), 1) + i * np.int32(_VT)
        row = lax.broadcasted_iota(jnp.int32, (_B, _VT), 0)
        cnt_ref[...] = row * np.int32(_V) + col

    cnt = cnt_ref[...]
    n = _normal(_threefry_xor(kn_ref[j, 0], kn_ref[j, 1], cnt))
    g = _gumbel(_threefry_xor(kg_ref[j, 0], kg_ref[j, 1], cnt))

    t = t_ref[j, 0]
    score = (logits_ref[...] + n * t) + g

    m = jnp.max(score, axis=1, keepdims=True)
    loc = lax.broadcasted_iota(jnp.int32, (_B, _VT), 1)
    cand = jnp.where(score == m, loc, np.int32(0x7FFFFFFF))
    ix = jnp.min(cand, axis=1, keepdims=True) + i * np.int32(_VT)

    val_ref[...] = m.reshape(1, 1, 1, _B)
    idx_ref[...] = ix.reshape(1, 1, 1, _B)


def _merge_mode_kernel(val_ref, idx_ref, out_ref):
    bv = val_ref[0, :, 0, :]
    bi = idx_ref[0, :, 0, :]
    for k in range(1, _NT):
        v = val_ref[k, :, 0, :]
        ixk = idx_ref[k, :, 0, :]
        better = v > bv
        bv = jnp.where(better, v, bv)
        bi = jnp.where(better, ixk, bi)
    counts = jnp.zeros((_NS, _B), dtype=jnp.int32)
    for s in range(_NS):
        counts = counts + (bi == bi[s:s + 1, :]).astype(jnp.int32)
    maxc = jnp.max(counts, axis=0, keepdims=True)
    mode = jnp.min(jnp.where(counts == maxc, bi, np.int32(_V)),
                   axis=0, keepdims=True)
    out_ref[...] = mode


def kernel(h, W, b):
    Wp = jnp.pad(W, ((0, _VPAD - _V), (0, 0)))
    bp = jnp.pad(b, (0, _VPAD - _V), constant_values=_NEG).reshape(1, _VPAD)

    root = jax.random.key(1)
    step_keys = jax.random.split(root, _NS)
    sub = jax.vmap(jax.random.split)(step_keys)
    kn = lax.bitcast_convert_type(jax.random.key_data(sub[:, 0]), jnp.int32)
    kg = lax.bitcast_convert_type(jax.random.key_data(sub[:, 1]), jnp.int32)
    temps = jnp.linspace(1.0, 0.01, _NS).astype(jnp.float32).reshape(_NS, 1)

    val, idx = pl.pallas_call(
        _sample_kernel,
        grid=(_NT, _NS),
        in_specs=[
            pl.BlockSpec((_B, _D), lambda i, j: (0, 0)),
            pl.BlockSpec((_VT, _D), lambda i, j: (i, 0)),
            pl.BlockSpec((1, _VT), lambda i, j: (0, i)),
            pl.BlockSpec((_NS, 2), lambda i, j: (0, 0), memory_space=pltpu.SMEM),
            pl.BlockSpec((_NS, 2), lambda i, j: (0, 0), memory_space=pltpu.SMEM),
            pl.BlockSpec((_NS, 1), lambda i, j: (0, 0), memory_space=pltpu.SMEM),
        ],
        out_specs=[
            pl.BlockSpec((1, 1, 1, _B), lambda i, j: (i, j, 0, 0)),
            pl.BlockSpec((1, 1, 1, _B), lambda i, j: (i, j, 0, 0)),
        ],
        out_shape=[
            jax.ShapeDtypeStruct((_NT, _NS, 1, _B), jnp.float32),
            jax.ShapeDtypeStruct((_NT, _NS, 1, _B), jnp.int32),
        ],
        scratch_shapes=[pltpu.VMEM((_B, _VT), jnp.float32),
                        pltpu.VMEM((_B, _VT), jnp.int32)],
    )(h, Wp, bp, kn, kg, temps)

    out = pl.pallas_call(
        _merge_mode_kernel,
        out_shape=jax.ShapeDtypeStruct((1, _B), jnp.int32),
    )(val, idx)
    return out.reshape(_B)


# speedup vs baseline: 1.1531x; 1.0405x over previous
import functools

import jax
import jax.numpy as jnp
import numpy as np
from jax import lax
from jax.experimental import pallas as pl
from jax.experimental.pallas import tpu as pltpu
from jax.experimental.pallas import tpu_sc as plsc

_V = 50257
_B = 64
_D = 2048
_NS = 100

_NW = 32
_CW = 64
_SCV = _NW * _CW
_VS = _V - _SCV

_VT = 2048
_NT = (_VS + _VT - 1) // _VT
_VPAD = _NT * _VT

_R0 = (13, 15, 26, 6)
_R1 = (17, 29, 16, 24)

_LO = np.float32(np.nextafter(np.float32(-1.0), np.float32(0.0)))
_TINY = np.float32(np.finfo(np.float32).tiny)
_SQRT2 = np.float32(np.sqrt(2.0))
_NEG = np.float32(-1e30)
_LN2 = np.float32(0.6931471805599453)


def _rotl(x, d):
    return lax.bitwise_or(lax.shift_left(x, np.int32(d)),
                          lax.shift_right_logical(x, np.int32(32 - d)))


def _rounds(x0, x1, rots):
    for r in rots:
        x0 = x0 + x1
        x1 = _rotl(x1, r)
        x1 = lax.bitwise_xor(x0, x1)
    return x0, x1


def _threefry_xor(k0, k1, c1):
    ks2 = lax.bitwise_xor(lax.bitwise_xor(k0, k1), np.int32(0x1BD11BDA))
    x0 = jnp.zeros_like(c1) + k0
    x1 = c1 + k1
    x0, x1 = _rounds(x0, x1, _R0); x0 = x0 + k1; x1 = x1 + ks2 + np.int32(1)
    x0, x1 = _rounds(x0, x1, _R1); x0 = x0 + ks2; x1 = x1 + k0 + np.int32(2)
    x0, x1 = _rounds(x0, x1, _R0); x0 = x0 + k0; x1 = x1 + k1 + np.int32(3)
    x0, x1 = _rounds(x0, x1, _R1); x0 = x0 + k1; x1 = x1 + ks2 + np.int32(4)
    x0, x1 = _rounds(x0, x1, _R0); x0 = x0 + ks2; x1 = x1 + k0 + np.int32(5)
    return lax.bitwise_xor(x0, x1)


def _u01(bits):
    fb = lax.bitwise_or(lax.shift_right_logical(bits, np.int32(9)),
                        np.int32(0x3F800000))
    return lax.bitcast_convert_type(fb, jnp.float32) - np.float32(1.0)


def _erfinv_polys(x, w, sq):
    small = w < np.float32(5.0)
    ws = w - np.float32(2.5)
    wb = sq - np.float32(3.0)
    p1 = np.float32(2.81022636e-08)
    for c in (3.43273939e-07, -3.5233877e-06, -4.39150654e-06, 0.00021858087,
              -0.00125372503, -0.00417768164, 0.246640727, 1.50140941):
        p1 = p1 * ws + np.float32(c)
    p2 = np.float32(-0.000200214257)
    for c in (0.000100950558, 0.00134934322, -0.00367342844, 0.00573950773,
              -0.0076224613, 0.00943887047, 1.00167406, 2.83297682):
        p2 = p2 * wb + np.float32(c)
    return jnp.where(small, p1, p2) * x


def _normal_tc(bits):
    f = _u01(bits)
    u = f * (np.float32(1.0) - _LO) + _LO
    u = jnp.maximum(_LO, u)
    w = -jnp.log1p(-u * u)
    return _SQRT2 * _erfinv_polys(u, w, jnp.sqrt(w))


def _gumbel_tc(bits):
    f = _u01(bits)
    u = f * (np.float32(1.0) - _TINY) + _TINY
    u = jnp.maximum(_TINY, u)
    return -jnp.log(-jnp.log(u))


def _sample_kernel(h_ref, w_ref, b_ref, kn_ref, kg_ref, t_ref,
                   val_ref, idx_ref, logits_ref, cnt_ref):
    i = pl.program_id(0)
    j = pl.program_id(1)

    @pl.when(j == 0)
    def _matmul():
        acc = lax.dot_general(h_ref[...], w_ref[...],
                              (((1,), (1,)), ((), ())),
                              preferred_element_type=jnp.float32)
        logits_ref[...] = acc + b_ref[0, :][None, :]
        col = lax.broadcasted_iota(jnp.int32, (_B, _VT), 1) + i * np.int32(_VT)
        row = lax.broadcasted_iota(jnp.int32, (_B, _VT), 0)
        cnt_ref[...] = row * np.int32(_V) + col

    cnt = cnt_ref[...]
    n = _normal_tc(_threefry_xor(kn_ref[j, 0], kn_ref[j, 1], cnt))
    g = _gumbel_tc(_threefry_xor(kg_ref[j, 0], kg_ref[j, 1], cnt))

    t = t_ref[j, 0]
    score = (logits_ref[...] + n * t) + g

    m = jnp.max(score, axis=1, keepdims=True)
    loc = lax.broadcasted_iota(jnp.int32, (_B, _VT), 1)
    cand = jnp.where(score == m, loc, np.int32(0x7FFFFFFF))
    ix = jnp.min(cand, axis=1, keepdims=True) + i * np.int32(_VT)

    val_ref[...] = m.reshape(1, 1, 1, _B)
    idx_ref[...] = ix.reshape(1, 1, 1, _B)


def _softlog(y):
    bits = lax.bitcast_convert_type(y, jnp.int32)
    e = lax.shift_right_logical(bits, np.int32(23)) - np.int32(127)
    m = lax.bitcast_convert_type(
        lax.bitwise_or(lax.bitwise_and(bits, np.int32(0x007FFFFF)),
                       np.int32(0x3F800000)), jnp.float32)
    big = m > np.float32(4.0 / 3.0)
    m = jnp.where(big, m * np.float32(0.5), m)
    e = e + jnp.where(big, np.int32(1), np.int32(0))
    z = (m - np.float32(1.0)) / (m + np.float32(1.0))
    z2 = z * z
    p = np.float32(1.0 / 9.0)
    for c in (1.0 / 7.0, 1.0 / 5.0, 1.0 / 3.0, 1.0):
        p = p * z2 + np.float32(c)
    return e.astype(jnp.float32) * _LN2 + np.float32(2.0) * z * p


def _normal_sc(bits):
    f = _u01(bits)
    u = f * (np.float32(1.0) - _LO) + _LO
    u = jnp.maximum(_LO, u)
    v = -u * u
    y = np.float32(1.0) + v
    w = -(_softlog(y) + (v - (y - np.float32(1.0))) / y)
    sq = jnp.exp(np.float32(0.5) * _softlog(w))
    return _SQRT2 * _erfinv_polys(u, w, sq)


def _gumbel_sc(bits):
    f = _u01(bits)
    u = f * (np.float32(1.0) - _TINY) + _TINY
    u = jnp.maximum(_TINY, u)
    return -_softlog(-_softlog(u))


def _sc_sampler_body(ltT_hbm, kn_hbm, kg_hbm, t_hbm, val_hbm, idx_hbm,
                     lt_v, kn_v, kg_v, t_v, val_v, idx_v):
    c = lax.axis_index("c")
    s = lax.axis_index("s")
    w = s * 2 + c
    colbase = _VS + w * _CW

    pltpu.sync_copy(ltT_hbm.at[pl.ds(w * _CW, _CW)], lt_v)
    pltpu.sync_copy(kn_hbm, kn_v)
    pltpu.sync_copy(kg_hbm, kg_v)
    pltpu.sync_copy(t_hbm, t_v)

    lane = lax.iota(jnp.int32, 16)
    rowbase = [lane * np.int32(_V) + np.int32(16 * r * _V) for r in range(4)]

    def step(j, carry):
        knr = kn_v[j]
        kgr = kg_v[j]
        kn0 = knr[0]
        kn1 = knr[1]
        kg0 = kgr[0]
        kg1 = kgr[1]
        t = t_v[j][0]
        for r in range(4):
            def col_body(cc, mi):
                vmax, vidx = mi
                colg = colbase + cc
                cnt = rowbase[r] + colg
                n = _normal_sc(_threefry_xor(kn0, kn1, cnt))
                g = _gumbel_sc(_threefry_xor(kg0, kg1, cnt))
                score = (lt_v[cc, pl.ds(16 * r, 16)] + n * t) + g
                upd = score > vmax
                vmax = jnp.where(upd, score, vmax)
                vidx = jnp.where(upd, jnp.full((16,), 1, jnp.int32) * colg,
                                 vidx)
                return vmax, vidx
            vmax0 = jnp.full((16,), _NEG, jnp.float32)
            vidx0 = jnp.zeros((16,), jnp.int32)
            vmax, vidx = lax.fori_loop(0, _CW, col_body, (vmax0, vidx0))
            val_v[j, pl.ds(16 * r, 16)] = vmax
            idx_v[j, pl.ds(16 * r, 16)] = vidx
        return carry

    lax.fori_loop(0, _NS, step, 0)

    pltpu.sync_copy(val_v, val_hbm.at[w])
    pltpu.sync_copy(idx_v, idx_hbm.at[w])


def _merge_mode_kernel(val_ref, idx_ref, vsc_ref, isc_ref, out_ref):
    bv = val_ref[0, :, 0, :]
    bi = idx_ref[0, :, 0, :]
    for k in range(1, _NT):
        v = val_ref[k, :, 0, :]
        ixk = idx_ref[k, :, 0, :]
        better = v > bv
        bv = jnp.where(better, v, bv)
        bi = jnp.where(better, ixk, bi)
    for k in range(_NW):
        v = vsc_ref[k]
        ixk = isc_ref[k]
        better = v > bv
        bv = jnp.where(better, v, bv)
        bi = jnp.where(better, ixk, bi)
    counts = jnp.zeros((_NS, _B), dtype=jnp.int32)
    for s in range(_NS):
        counts = counts + (bi == bi[s:s + 1, :]).astype(jnp.int32)
    maxc = jnp.max(counts, axis=0, keepdims=True)
    mode = jnp.min(jnp.where(counts == maxc, bi, np.int32(_V)),
                   axis=0, keepdims=True)
    out_ref[...] = mode


def _mm_sc_kernel(w_ref, h_ref, b_ref, out_ref):
    acc = lax.dot_general(w_ref[...], h_ref[...],
                          (((1,), (1,)), ((), ())),
                          preferred_element_type=jnp.float32)
    out_ref[...] = acc + b_ref[0, :][:, None]


def kernel(h, W, b):
    Wp = jnp.pad(W[:_VS], ((0, _VPAD - _VS), (0, 0)))
    bp = jnp.pad(b[:_VS], (0, _VPAD - _VS), constant_values=_NEG).reshape(1, _VPAD)

    root = jax.random.key(1)
    step_keys = jax.random.split(root, _NS)
    sub = jax.vmap(jax.random.split)(step_keys)
    kn = lax.bitcast_convert_type(jax.random.key_data(sub[:, 0]), jnp.int32)
    kg = lax.bitcast_convert_type(jax.random.key_data(sub[:, 1]), jnp.int32)
    temps = jnp.linspace(1.0, 0.01, _NS).astype(jnp.float32)

    ltT = pl.pallas_call(
        _mm_sc_kernel,
        out_shape=jax.ShapeDtypeStruct((_SCV, _B), jnp.float32),
    )(W[_VS:], h, b[_VS:].reshape(1, _SCV))

    sc = pl.kernel(
        _sc_sampler_body,
        mesh=plsc.VectorSubcoreMesh(core_axis_name="c", subcore_axis_name="s"),
        out_type=[jax.ShapeDtypeStruct((_NW, _NS, _B), jnp.float32),
                  jax.ShapeDtypeStruct((_NW, _NS, _B), jnp.int32)],
        scratch_types=[pltpu.VMEM((_CW, _B), jnp.float32),
                       pltpu.VMEM((_NS, 16), jnp.int32),
                       pltpu.VMEM((_NS, 16), jnp.int32),
                       pltpu.VMEM((_NS, 16), jnp.float32),
                       pltpu.VMEM((_NS, _B), jnp.float32),
                       pltpu.VMEM((_NS, _B), jnp.int32)],
    )
    kn16 = jnp.pad(kn, ((0, 0), (0, 14)))
    kg16 = jnp.pad(kg, ((0, 0), (0, 14)))
    t16 = jnp.tile(temps.reshape(_NS, 1), (1, 16))
    val_sc, idx_sc = sc(ltT, kn16, kg16, t16)

    val, idx = pl.pallas_call(
        _sample_kernel,
        grid=(_NT, _NS),
        in_specs=[
            pl.BlockSpec((_B, _D), lambda i, j: (0, 0)),
            pl.BlockSpec((_VT, _D), lambda i, j: (i, 0)),
            pl.BlockSpec((1, _VT), lambda i, j: (0, i)),
            pl.BlockSpec((_NS, 2), lambda i, j: (0, 0), memory_space=pltpu.SMEM),
            pl.BlockSpec((_NS, 2), lambda i, j: (0, 0), memory_space=pltpu.SMEM),
            pl.BlockSpec((_NS, 1), lambda i, j: (0, 0), memory_space=pltpu.SMEM),
        ],
        out_specs=[
            pl.BlockSpec((1, 1, 1, _B), lambda i, j: (i, j, 0, 0)),
            pl.BlockSpec((1, 1, 1, _B), lambda i, j: (i, j, 0, 0)),
        ],
        out_shape=[
            jax.ShapeDtypeStruct((_NT, _NS, 1, _B), jnp.float32),
            jax.ShapeDtypeStruct((_NT, _NS, 1, _B), jnp.int32),
        ],
        scratch_shapes=[pltpu.VMEM((_B, _VT), jnp.float32),
                        pltpu.VMEM((_B, _VT), jnp.int32)],
    )(h, Wp, bp, kn, kg, temps.reshape(_NS, 1))

    out = pl.pallas_call(
        _merge_mode_kernel,
        out_shape=jax.ShapeDtypeStruct((1, _B), jnp.int32),
    )(val, idx, val_sc, idx_sc)
    return out.reshape(_B)


# speedup vs baseline: 1.4455x; 1.2536x over previous
import functools

import jax
import jax.numpy as jnp
import numpy as np
from jax import lax
from jax.experimental import pallas as pl
from jax.experimental.pallas import tpu as pltpu
from jax.experimental.pallas import tpu_sc as plsc

_V = 50257
_B = 64
_D = 2048
_NS = 100

_NW = 32
_CW = 360
_SCV = _NW * _CW
_VS = _V - _SCV

_VT = 2048
_NT = (_VS + _VT - 1) // _VT
_VPAD = _NT * _VT

_R0 = (13, 15, 26, 6)
_R1 = (17, 29, 16, 24)

_LO = np.float32(np.nextafter(np.float32(-1.0), np.float32(0.0)))
_TINY = np.float32(np.finfo(np.float32).tiny)
_SQRT2 = np.float32(np.sqrt(2.0))
_NEG = np.float32(-1e30)
_LN2 = np.float32(0.6931471805599453)


def _rotl(x, d):
    return lax.bitwise_or(lax.shift_left(x, np.int32(d)),
                          lax.shift_right_logical(x, np.int32(32 - d)))


def _rounds(x0, x1, rots):
    for r in rots:
        x0 = x0 + x1
        x1 = _rotl(x1, r)
        x1 = lax.bitwise_xor(x0, x1)
    return x0, x1


def _threefry_xor(k0, k1, c1):
    ks2 = lax.bitwise_xor(lax.bitwise_xor(k0, k1), np.int32(0x1BD11BDA))
    x0 = jnp.zeros_like(c1) + k0
    x1 = c1 + k1
    x0, x1 = _rounds(x0, x1, _R0); x0 = x0 + k1; x1 = x1 + ks2 + np.int32(1)
    x0, x1 = _rounds(x0, x1, _R1); x0 = x0 + ks2; x1 = x1 + k0 + np.int32(2)
    x0, x1 = _rounds(x0, x1, _R0); x0 = x0 + k0; x1 = x1 + k1 + np.int32(3)
    x0, x1 = _rounds(x0, x1, _R1); x0 = x0 + k1; x1 = x1 + ks2 + np.int32(4)
    x0, x1 = _rounds(x0, x1, _R0); x0 = x0 + ks2; x1 = x1 + k0 + np.int32(5)
    return lax.bitwise_xor(x0, x1)


def _u01(bits):
    fb = lax.bitwise_or(lax.shift_right_logical(bits, np.int32(9)),
                        np.int32(0x3F800000))
    return lax.bitcast_convert_type(fb, jnp.float32) - np.float32(1.0)


def _erfinv_polys(x, w, sq):
    small = w < np.float32(5.0)
    ws = w - np.float32(2.5)
    wb = sq - np.float32(3.0)
    p1 = np.float32(2.81022636e-08)
    for c in (3.43273939e-07, -3.5233877e-06, -4.39150654e-06, 0.00021858087,
              -0.00125372503, -0.00417768164, 0.246640727, 1.50140941):
        p1 = p1 * ws + np.float32(c)
    p2 = np.float32(-0.000200214257)
    for c in (0.000100950558, 0.00134934322, -0.00367342844, 0.00573950773,
              -0.0076224613, 0.00943887047, 1.00167406, 2.83297682):
        p2 = p2 * wb + np.float32(c)
    return jnp.where(small, p1, p2) * x


def _normal_tc(bits):
    f = _u01(bits)
    u = f * (np.float32(1.0) - _LO) + _LO
    u = jnp.maximum(_LO, u)
    w = -jnp.log1p(-u * u)
    return _SQRT2 * _erfinv_polys(u, w, jnp.sqrt(w))


def _gumbel_tc(bits):
    f = _u01(bits)
    u = f * (np.float32(1.0) - _TINY) + _TINY
    u = jnp.maximum(_TINY, u)
    return -jnp.log(-jnp.log(u))


def _sample_kernel(h_ref, w_ref, b_ref, kn_ref, kg_ref, t_ref,
                   val_ref, idx_ref, logits_ref, cnt_ref):
    i = pl.program_id(0)
    j = pl.program_id(1)

    @pl.when(j == 0)
    def _matmul():
        acc = lax.dot_general(h_ref[...], w_ref[...],
                              (((1,), (1,)), ((), ())),
                              preferred_element_type=jnp.float32)
        logits_ref[...] = acc + b_ref[0, :][None, :]
        col = lax.broadcasted_iota(jnp.int32, (_B, _VT), 1) + i * np.int32(_VT)
        row = lax.broadcasted_iota(jnp.int32, (_B, _VT), 0)
        cnt_ref[...] = row * np.int32(_V) + col

    cnt = cnt_ref[...]
    n = _normal_tc(_threefry_xor(kn_ref[j, 0], kn_ref[j, 1], cnt))
    g = _gumbel_tc(_threefry_xor(kg_ref[j, 0], kg_ref[j, 1], cnt))

    t = t_ref[j, 0]
    score = (logits_ref[...] + n * t) + g

    m = jnp.max(score, axis=1, keepdims=True)
    loc = lax.broadcasted_iota(jnp.int32, (_B, _VT), 1)
    cand = jnp.where(score == m, loc, np.int32(0x7FFFFFFF))
    ix = jnp.min(cand, axis=1, keepdims=True) + i * np.int32(_VT)

    val_ref[...] = m.reshape(1, 1, 1, _B)
    idx_ref[...] = ix.reshape(1, 1, 1, _B)


def _softlog(y):
    bits = lax.bitcast_convert_type(y, jnp.int32)
    e = lax.shift_right_logical(bits, np.int32(23)) - np.int32(127)
    m = lax.bitcast_convert_type(
        lax.bitwise_or(lax.bitwise_and(bits, np.int32(0x007FFFFF)),
                       np.int32(0x3F800000)), jnp.float32)
    big = m > np.float32(4.0 / 3.0)
    m = jnp.where(big, m * np.float32(0.5), m)
    e = e + jnp.where(big, np.int32(1), np.int32(0))
    z = (m - np.float32(1.0)) / (m + np.float32(1.0))
    z2 = z * z
    p = np.float32(1.0 / 9.0)
    for c in (1.0 / 7.0, 1.0 / 5.0, 1.0 / 3.0, 1.0):
        p = p * z2 + np.float32(c)
    return e.astype(jnp.float32) * _LN2 + np.float32(2.0) * z * p


def _normal_sc(bits):
    f = _u01(bits)
    u = f * (np.float32(1.0) - _LO) + _LO
    u = jnp.maximum(_LO, u)
    v = -u * u
    y = np.float32(1.0) + v
    w = -(_softlog(y) + (v - (y - np.float32(1.0))) / y)
    sq = jnp.exp(np.float32(0.5) * _softlog(w))
    return _SQRT2 * _erfinv_polys(u, w, sq)


def _gumbel_sc(bits):
    f = _u01(bits)
    u = f * (np.float32(1.0) - _TINY) + _TINY
    u = jnp.maximum(_TINY, u)
    return -_softlog(-_softlog(u))


def _sc_sampler_body(ltT_hbm, kn_hbm, kg_hbm, t_hbm, val_hbm, idx_hbm,
                     lt_v, kn_v, kg_v, t_v, val_v, idx_v):
    c = lax.axis_index("c")
    s = lax.axis_index("s")
    w = s * 2 + c
    colbase = _VS + w * _CW

    pltpu.sync_copy(ltT_hbm.at[pl.ds(w * _CW, _CW)], lt_v)
    pltpu.sync_copy(kn_hbm, kn_v)
    pltpu.sync_copy(kg_hbm, kg_v)
    pltpu.sync_copy(t_hbm, t_v)

    lane = lax.iota(jnp.int32, 16)
    rowbase = [lane * np.int32(_V) + np.int32(16 * r * _V) for r in range(4)]

    def step(j, carry):
        knr = kn_v[j]
        kgr = kg_v[j]
        kn0 = knr[0]
        kn1 = knr[1]
        kg0 = kgr[0]
        kg1 = kgr[1]
        t = t_v[j][0]
        for r in range(4):
            def col_body(cc, mi):
                vmax, vidx = mi
                colg = colbase + cc
                cnt = rowbase[r] + colg
                n = _normal_sc(_threefry_xor(kn0, kn1, cnt))
                g = _gumbel_sc(_threefry_xor(kg0, kg1, cnt))
                score = (lt_v[cc, pl.ds(16 * r, 16)] + n * t) + g
                upd = score > vmax
                vmax = jnp.where(upd, score, vmax)
                vidx = jnp.where(upd, jnp.full((16,), 1, jnp.int32) * colg,
                                 vidx)
                return vmax, vidx
            vmax0 = jnp.full((16,), _NEG, jnp.float32)
            vidx0 = jnp.zeros((16,), jnp.int32)
            vmax, vidx = lax.fori_loop(0, _CW, col_body, (vmax0, vidx0))
            val_v[j, pl.ds(16 * r, 16)] = vmax
            idx_v[j, pl.ds(16 * r, 16)] = vidx
        return carry

    lax.fori_loop(0, _NS, step, 0)

    pltpu.sync_copy(val_v, val_hbm.at[w])
    pltpu.sync_copy(idx_v, idx_hbm.at[w])


def _merge_mode_kernel(val_ref, idx_ref, vsc_ref, isc_ref, out_ref):
    bv = val_ref[0, :, 0, :]
    bi = idx_ref[0, :, 0, :]
    for k in range(1, _NT):
        v = val_ref[k, :, 0, :]
        ixk = idx_ref[k, :, 0, :]
        better = v > bv
        bv = jnp.where(better, v, bv)
        bi = jnp.where(better, ixk, bi)
    for k in range(_NW):
        v = vsc_ref[k]
        ixk = isc_ref[k]
        better = v > bv
        bv = jnp.where(better, v, bv)
        bi = jnp.where(better, ixk, bi)
    counts = jnp.zeros((_NS, _B), dtype=jnp.int32)
    for s in range(_NS):
        counts = counts + (bi == bi[s:s + 1, :]).astype(jnp.int32)
    maxc = jnp.max(counts, axis=0, keepdims=True)
    mode = jnp.min(jnp.where(counts == maxc, bi, np.int32(_V)),
                   axis=0, keepdims=True)
    out_ref[...] = mode


def _mm_sc_kernel(w_ref, h_ref, b_ref, out_ref):
    acc = lax.dot_general(w_ref[...], h_ref[...],
                          (((1,), (1,)), ((), ())),
                          preferred_element_type=jnp.float32)
    out_ref[...] = acc + b_ref[0, :][:, None]


def kernel(h, W, b):
    Wp = jnp.pad(W[:_VS], ((0, _VPAD - _VS), (0, 0)))
    bp = jnp.pad(b[:_VS], (0, _VPAD - _VS), constant_values=_NEG).reshape(1, _VPAD)

    root = jax.random.key(1)
    step_keys = jax.random.split(root, _NS)
    sub = jax.vmap(jax.random.split)(step_keys)
    kn = lax.bitcast_convert_type(jax.random.key_data(sub[:, 0]), jnp.int32)
    kg = lax.bitcast_convert_type(jax.random.key_data(sub[:, 1]), jnp.int32)
    temps = jnp.linspace(1.0, 0.01, _NS).astype(jnp.float32)

    _GM = 10
    _MT = _SCV // _GM
    ltT = pl.pallas_call(
        _mm_sc_kernel,
        grid=(_GM,),
        in_specs=[
            pl.BlockSpec((_MT, _D), lambda i: (i, 0)),
            pl.BlockSpec((_B, _D), lambda i: (0, 0)),
            pl.BlockSpec((1, _MT), lambda i: (0, i)),
        ],
        out_specs=pl.BlockSpec((_MT, _B), lambda i: (i, 0)),
        out_shape=jax.ShapeDtypeStruct((_SCV, _B), jnp.float32),
    )(W[_VS:], h, b[_VS:].reshape(1, _SCV))

    sc = pl.kernel(
        _sc_sampler_body,
        mesh=plsc.VectorSubcoreMesh(core_axis_name="c", subcore_axis_name="s"),
        out_type=[jax.ShapeDtypeStruct((_NW, _NS, _B), jnp.float32),
                  jax.ShapeDtypeStruct((_NW, _NS, _B), jnp.int32)],
        scratch_types=[pltpu.VMEM((_CW, _B), jnp.float32),
                       pltpu.VMEM((_NS, 16), jnp.int32),
                       pltpu.VMEM((_NS, 16), jnp.int32),
                       pltpu.VMEM((_NS, 16), jnp.float32),
                       pltpu.VMEM((_NS, _B), jnp.float32),
                       pltpu.VMEM((_NS, _B), jnp.int32)],
    )
    kn16 = jnp.pad(kn, ((0, 0), (0, 14)))
    kg16 = jnp.pad(kg, ((0, 0), (0, 14)))
    t16 = jnp.tile(temps.reshape(_NS, 1), (1, 16))
    val_sc, idx_sc = sc(ltT, kn16, kg16, t16)

    val, idx = pl.pallas_call(
        _sample_kernel,
        grid=(_NT, _NS),
        in_specs=[
            pl.BlockSpec((_B, _D), lambda i, j: (0, 0)),
            pl.BlockSpec((_VT, _D), lambda i, j: (i, 0)),
            pl.BlockSpec((1, _VT), lambda i, j: (0, i)),
            pl.BlockSpec((_NS, 2), lambda i, j: (0, 0), memory_space=pltpu.SMEM),
            pl.BlockSpec((_NS, 2), lambda i, j: (0, 0), memory_space=pltpu.SMEM),
            pl.BlockSpec((_NS, 1), lambda i, j: (0, 0), memory_space=pltpu.SMEM),
        ],
        out_specs=[
            pl.BlockSpec((1, 1, 1, _B), lambda i, j: (i, j, 0, 0)),
            pl.BlockSpec((1, 1, 1, _B), lambda i, j: (i, j, 0, 0)),
        ],
        out_shape=[
            jax.ShapeDtypeStruct((_NT, _NS, 1, _B), jnp.float32),
            jax.ShapeDtypeStruct((_NT, _NS, 1, _B), jnp.int32),
        ],
        scratch_shapes=[pltpu.VMEM((_B, _VT), jnp.float32),
                        pltpu.VMEM((_B, _VT), jnp.int32)],
    )(h, Wp, bp, kn, kg, temps.reshape(_NS, 1))

    out = pl.pallas_call(
        _merge_mode_kernel,
        out_shape=jax.ShapeDtypeStruct((1, _B), jnp.int32),
    )(val, idx, val_sc, idx_sc)
    return out.reshape(_B)
